# Initial kernel scaffold; baseline (speedup 1.0000x reference)
import jax
import jax.numpy as jnp
from jax.experimental import pallas as pl


def kernel(uv_map, texture0):
    raise NotImplementedError("write your pallas kernel here")


# speedup vs baseline: 1.8075x; 1.8075x over previous
import functools

import jax
import jax.numpy as jnp
from jax import lax
from jax.experimental import pallas as pl
from jax.experimental.pallas import tpu as pltpu
from jax.experimental.pallas import tpu_sc as plsc

NC, NS, L = 2, 16, 16
NW = NC * NS
TEX = 1024
C = 16
N, H, W = 4, 512, 512
ROWS = N * H
ROWS_PER_W = ROWS // NW
B = W
NG = B // L
NSUB = B // 128


def _floor_i32(x):
    t = x.astype(jnp.int32)
    tf = t.astype(jnp.float32)
    return t - jnp.where(x < tf, 1, 0).astype(jnp.int32)


def _tex_kernel(uv_hbm, tex_hbm, out_hbm, uv_v, w_v, idx_v, rows_v, acc_v,
                sem_g, sem_o):
    wid = lax.axis_index("s") * NC + lax.axis_index("c")
    lanes = lax.iota(jnp.int32, L)

    def chunk_body(i, carry):
        r = wid * ROWS_PER_W + i
        pltpu.sync_copy(uv_hbm.at[r], uv_v)

        def idx_body(g, c2):
            pix = g * L + lanes
            u = plsc.load_gather(uv_v, [pix * 2])
            v = plsc.load_gather(uv_v, [pix * 2 + 1])
            gx = u * 2.0 - 1.0
            gy = -(v * 2.0 - 1.0)
            fx = ((gx + 1.0) * float(TEX) - 1.0) * 0.5
            fy = ((gy + 1.0) * float(TEX) - 1.0) * 0.5
            ix0 = _floor_i32(fx)
            iy0 = _floor_i32(fy)
            wx1 = fx - ix0.astype(jnp.float32)
            wy1 = fy - iy0.astype(jnp.float32)
            wx0 = 1.0 - wx1
            wy0 = 1.0 - wy1
            ix1 = ix0 + 1
            iy1 = iy0 + 1
            m = jnp.where(u == 0.0, 0.0, 1.0)
            wx0 = wx0 * m * jnp.where((ix0 >= 0) & (ix0 <= TEX - 1), 1.0, 0.0)
            wx1 = wx1 * m * jnp.where((ix1 >= 0) & (ix1 <= TEX - 1), 1.0, 0.0)
            wy0 = wy0 * jnp.where((iy0 >= 0) & (iy0 <= TEX - 1), 1.0, 0.0)
            wy1 = wy1 * jnp.where((iy1 >= 0) & (iy1 <= TEX - 1), 1.0, 0.0)
            cx0 = jnp.clip(ix0, 0, TEX - 1)
            cx1 = jnp.clip(ix1, 0, TEX - 1)
            rb0 = jnp.clip(iy0, 0, TEX - 1) * TEX
            rb1 = jnp.clip(iy1, 0, TEX - 1) * TEX
            off = g * L
            idx_v[pl.ds(0 * B + off, L)] = rb0 + cx0
            idx_v[pl.ds(1 * B + off, L)] = rb0 + cx1
            idx_v[pl.ds(2 * B + off, L)] = rb1 + cx0
            idx_v[pl.ds(3 * B + off, L)] = rb1 + cx1
            w_v[pl.ds(0 * B + off, L)] = wy0 * wx0
            w_v[pl.ds(1 * B + off, L)] = wy0 * wx1
            w_v[pl.ds(2 * B + off, L)] = wy1 * wx0
            w_v[pl.ds(3 * B + off, L)] = wy1 * wx1
            return c2

        lax.fori_loop(0, NG, idx_body, 0)

        copies = []
        for k in range(4):
            for s in range(NSUB):
                seg = k * NSUB + s
                cp = pltpu.async_copy(
                    tex_hbm.at[idx_v.at[pl.ds(seg * 128, 128)]],
                    rows_v.at[seg], sem_g)
                copies.append(cp)
        for cp in copies:
            cp.wait()

        def comb_body(g, c2):
            s = g // 8
            jvec = (g - s * 8) * L + lanes
            off = g * L
            w00 = w_v[pl.ds(0 * B + off, L)]
            w01 = w_v[pl.ds(1 * B + off, L)]
            w10 = w_v[pl.ds(2 * B + off, L)]
            w11 = w_v[pl.ds(3 * B + off, L)]
            zc = jnp.zeros((L,), jnp.int32)
            k0 = zc + (0 * NSUB + s)
            k1 = zc + (1 * NSUB + s)
            k2 = zc + (2 * NSUB + s)
            k3 = zc + (3 * NSUB + s)
            for c in range(C):
                cvec = zc + c
                v00 = plsc.load_gather(rows_v, [k0, jvec, cvec])
                v01 = plsc.load_gather(rows_v, [k1, jvec, cvec])
                v10 = plsc.load_gather(rows_v, [k2, jvec, cvec])
                v11 = plsc.load_gather(rows_v, [k3, jvec, cvec])
                acc = v00 * w00 + v01 * w01 + v10 * w10 + v11 * w11
                acc_v[pl.ds(c * B + off, L)] = acc
            return c2

        lax.fori_loop(0, NG, comb_body, 0)

        ocopies = []
        for c in range(C):
            orow = r + (r // H) * (C - 1) * H + c * H
            cp = pltpu.async_copy(acc_v.at[pl.ds(c * B, B)], out_hbm.at[orow],
                                  sem_o)
            ocopies.append(cp)
        for cp in ocopies:
            cp.wait()
        return carry

    lax.fori_loop(0, ROWS_PER_W, chunk_body, 0)


@jax.jit
def kernel(uv_map, texture0):
    uv2 = uv_map.reshape(ROWS, 2 * W)
    tex2 = texture0.reshape(TEX * TEX, C)
    mesh = plsc.VectorSubcoreMesh(core_axis_name="c", subcore_axis_name="s")
    f = functools.partial(
        pl.kernel,
        out_type=jax.ShapeDtypeStruct((N * C * H, W), jnp.float32),
        mesh=mesh,
        compiler_params=pltpu.CompilerParams(needs_layout_passes=False,
                                             use_tc_tiling_on_sc=False),
        scratch_types=[
            pltpu.VMEM((2 * B,), jnp.float32),
            pltpu.VMEM((4 * B,), jnp.float32),
            pltpu.VMEM((4 * B,), jnp.int32),
            pltpu.VMEM((4 * NSUB, 128, C), jnp.float32),
            pltpu.VMEM((C * B,), jnp.float32),
            pltpu.SemaphoreType.DMA,
            pltpu.SemaphoreType.DMA,
        ],
    )(_tex_kernel)
    out2 = f(uv2, tex2)
    return out2.reshape(N, C, H, W)


# speedup vs baseline: 2.0366x; 1.1268x over previous
import functools

import jax
import jax.numpy as jnp
from jax import lax
from jax.experimental import pallas as pl
from jax.experimental.pallas import tpu as pltpu
from jax.experimental.pallas import tpu_sc as plsc

NC, NS, L = 2, 16, 16
NW = NC * NS
TEX = 1024
C = 16
N, H, W = 4, 512, 512
ROWS = N * H
ROWS_PER_W = ROWS // NW
B = W
NG = B // L
NSUB = B // 128
NSEG = 4 * NSUB


def _floor_i32(x):
    t = x.astype(jnp.int32)
    tf = t.astype(jnp.float32)
    return t - jnp.where(x < tf, 1, 0).astype(jnp.int32)


def _tex_kernel(uv_hbm, tex_hbm, out_hbm, uv_v, w0_v, w1_v, idx0_v, idx1_v,
                rows0_v, rows1_v, acc_v, semg0, semg1, semo):
    wid = lax.axis_index("s") * NC + lax.axis_index("c")
    lanes = lax.iota(jnp.int32, L)

    def fire_chunk(i, w_v, idx_v, rows_v, semg):
        r = wid * ROWS_PER_W + i
        pltpu.sync_copy(uv_hbm.at[r], uv_v)

        def idx_body(g, c2):
            pix = g * L + lanes
            u = plsc.load_gather(uv_v, [pix * 2])
            v = plsc.load_gather(uv_v, [pix * 2 + 1])
            gx = u * 2.0 - 1.0
            gy = -(v * 2.0 - 1.0)
            fx = ((gx + 1.0) * float(TEX) - 1.0) * 0.5
            fy = ((gy + 1.0) * float(TEX) - 1.0) * 0.5
            ix0 = _floor_i32(fx)
            iy0 = _floor_i32(fy)
            wx1 = fx - ix0.astype(jnp.float32)
            wy1 = fy - iy0.astype(jnp.float32)
            wx0 = 1.0 - wx1
            wy0 = 1.0 - wy1
            ix1 = ix0 + 1
            iy1 = iy0 + 1
            m = jnp.where(u == 0.0, 0.0, 1.0)
            wx0 = wx0 * m * jnp.where((ix0 >= 0) & (ix0 <= TEX - 1), 1.0, 0.0)
            wx1 = wx1 * m * jnp.where((ix1 >= 0) & (ix1 <= TEX - 1), 1.0, 0.0)
            wy0 = wy0 * jnp.where((iy0 >= 0) & (iy0 <= TEX - 1), 1.0, 0.0)
            wy1 = wy1 * jnp.where((iy1 >= 0) & (iy1 <= TEX - 1), 1.0, 0.0)
            cx0 = jnp.clip(ix0, 0, TEX - 1)
            cx1 = jnp.clip(ix1, 0, TEX - 1)
            rb0 = jnp.clip(iy0, 0, TEX - 1) * TEX
            rb1 = jnp.clip(iy1, 0, TEX - 1) * TEX
            off = g * L
            idx_v[pl.ds(0 * B + off, L)] = rb0 + cx0
            idx_v[pl.ds(1 * B + off, L)] = rb0 + cx1
            idx_v[pl.ds(2 * B + off, L)] = rb1 + cx0
            idx_v[pl.ds(3 * B + off, L)] = rb1 + cx1
            w_v[pl.ds(0 * B + off, L)] = wy0 * wx0
            w_v[pl.ds(1 * B + off, L)] = wy0 * wx1
            w_v[pl.ds(2 * B + off, L)] = wy1 * wx0
            w_v[pl.ds(3 * B + off, L)] = wy1 * wx1
            return c2

        lax.fori_loop(0, NG, idx_body, 0)
        for seg in range(NSEG):
            pltpu.async_copy(tex_hbm.at[idx_v.at[pl.ds(seg * 128, 128)]],
                             rows_v.at[seg], semg)

    def drain_gathers(idx_v, rows_v, semg):
        for seg in range(NSEG):
            pltpu.make_async_copy(
                tex_hbm.at[idx_v.at[pl.ds(seg * 128, 128)]],
                rows_v.at[seg], semg).wait()

    def combine_out(i, w_v, rows_v):
        r = wid * ROWS_PER_W + i

        def comb_body(g, c2):
            s = g // 8
            jvec = (g - s * 8) * L + lanes
            off = g * L
            w00 = w_v[pl.ds(0 * B + off, L)]
            w01 = w_v[pl.ds(1 * B + off, L)]
            w10 = w_v[pl.ds(2 * B + off, L)]
            w11 = w_v[pl.ds(3 * B + off, L)]
            zc = jnp.zeros((L,), jnp.int32)
            k0 = zc + (0 * NSUB + s)
            k1 = zc + (1 * NSUB + s)
            k2 = zc + (2 * NSUB + s)
            k3 = zc + (3 * NSUB + s)
            for c in range(C):
                cvec = zc + c
                v00 = plsc.load_gather(rows_v, [k0, jvec, cvec])
                v01 = plsc.load_gather(rows_v, [k1, jvec, cvec])
                v10 = plsc.load_gather(rows_v, [k2, jvec, cvec])
                v11 = plsc.load_gather(rows_v, [k3, jvec, cvec])
                acc = v00 * w00 + v01 * w01 + v10 * w10 + v11 * w11
                acc_v[pl.ds(c * B + off, L)] = acc
            return c2

        lax.fori_loop(0, NG, comb_body, 0)

        ocopies = []
        for c in range(C):
            orow = r + (r // H) * (C - 1) * H + c * H
            ocopies.append(pltpu.async_copy(
                acc_v.at[pl.ds(c * B, B)], out_hbm.at[orow], semo))
        for cp in ocopies:
            cp.wait()

    fire_chunk(0, w0_v, idx0_v, rows0_v, semg0)

    def body(ii, carry):
        c0 = 2 * ii
        c1 = c0 + 1
        fire_chunk(c1, w1_v, idx1_v, rows1_v, semg1)
        drain_gathers(idx0_v, rows0_v, semg0)
        combine_out(c0, w0_v, rows0_v)
        cn = jnp.minimum(c0 + 2, ROWS_PER_W - 1)
        fire_chunk(cn, w0_v, idx0_v, rows0_v, semg0)
        drain_gathers(idx1_v, rows1_v, semg1)
        combine_out(c1, w1_v, rows1_v)
        return carry

    lax.fori_loop(0, ROWS_PER_W // 2, body, 0)
    drain_gathers(idx0_v, rows0_v, semg0)


@jax.jit
def kernel(uv_map, texture0):
    uv2 = uv_map.reshape(ROWS, 2 * W)
    tex2 = texture0.reshape(TEX * TEX, C)
    mesh = plsc.VectorSubcoreMesh(core_axis_name="c", subcore_axis_name="s")
    f = functools.partial(
        pl.kernel,
        out_type=jax.ShapeDtypeStruct((N * C * H, W), jnp.float32),
        mesh=mesh,
        compiler_params=pltpu.CompilerParams(needs_layout_passes=False,
                                             use_tc_tiling_on_sc=False),
        scratch_types=[
            pltpu.VMEM((2 * B,), jnp.float32),
            pltpu.VMEM((4 * B,), jnp.float32),
            pltpu.VMEM((4 * B,), jnp.float32),
            pltpu.VMEM((4 * B,), jnp.int32),
            pltpu.VMEM((4 * B,), jnp.int32),
            pltpu.VMEM((NSEG, 128, C), jnp.float32),
            pltpu.VMEM((NSEG, 128, C), jnp.float32),
            pltpu.VMEM((C * B,), jnp.float32),
            pltpu.SemaphoreType.DMA,
            pltpu.SemaphoreType.DMA,
            pltpu.SemaphoreType.DMA,
        ],
    )(_tex_kernel)
    out2 = f(uv2, tex2)
    return out2.reshape(N, C, H, W)


# speedup vs baseline: 3.3352x; 1.6376x over previous
import functools

import jax
import jax.numpy as jnp
from jax import lax
from jax.experimental import pallas as pl
from jax.experimental.pallas import tpu as pltpu
from jax.experimental.pallas import tpu_sc as plsc

NC, NS, L = 2, 16, 16
NW = NC * NS
TEX = 1024
C = 16
N, H, W = 4, 512, 512
ROWS = N * H
ROWS_PER_W = ROWS // NW
B = W
NG = B // L
NSUB = B // 128
NSEG = 4 * NSUB


def _floor_i32(x):
    t = x.astype(jnp.int32)
    tf = t.astype(jnp.float32)
    return t - jnp.where(x < tf, 1, 0).astype(jnp.int32)


def _tex_kernel(uv_hbm, tex_hbm, out_hbm, uv_v, w0_v, w1_v, idx0_v, idx1_v,
                rows0_v, rows1_v, acc_v, semg0, semg1, semo):
    wid = lax.axis_index("s") * NC + lax.axis_index("c")
    lanes = lax.iota(jnp.int32, L)

    def fire_chunk(i, w_v, idx_v, rows_v, semg):
        r = wid * ROWS_PER_W + i
        pltpu.sync_copy(uv_hbm.at[r], uv_v)

        @plsc.parallel_loop(0, NG, 1, unroll=2)
        def idx_body(g):
            pix = g * L + lanes
            u = plsc.load_gather(uv_v, [pix * 2])
            v = plsc.load_gather(uv_v, [pix * 2 + 1])
            gx = u * 2.0 - 1.0
            gy = -(v * 2.0 - 1.0)
            fx = ((gx + 1.0) * float(TEX) - 1.0) * 0.5
            fy = ((gy + 1.0) * float(TEX) - 1.0) * 0.5
            ix0 = _floor_i32(fx)
            iy0 = _floor_i32(fy)
            wx1 = fx - ix0.astype(jnp.float32)
            wy1 = fy - iy0.astype(jnp.float32)
            wx0 = 1.0 - wx1
            wy0 = 1.0 - wy1
            ix1 = ix0 + 1
            iy1 = iy0 + 1
            m = jnp.where(u == 0.0, 0.0, 1.0)
            wx0 = wx0 * m * jnp.where((ix0 >= 0) & (ix0 <= TEX - 1), 1.0, 0.0)
            wx1 = wx1 * m * jnp.where((ix1 >= 0) & (ix1 <= TEX - 1), 1.0, 0.0)
            wy0 = wy0 * jnp.where((iy0 >= 0) & (iy0 <= TEX - 1), 1.0, 0.0)
            wy1 = wy1 * jnp.where((iy1 >= 0) & (iy1 <= TEX - 1), 1.0, 0.0)
            cx0 = jnp.clip(ix0, 0, TEX - 1)
            cx1 = jnp.clip(ix1, 0, TEX - 1)
            rb0 = jnp.clip(iy0, 0, TEX - 1) * TEX
            rb1 = jnp.clip(iy1, 0, TEX - 1) * TEX
            off = g * L
            idx_v[pl.ds(0 * B + off, L)] = rb0 + cx0
            idx_v[pl.ds(1 * B + off, L)] = rb0 + cx1
            idx_v[pl.ds(2 * B + off, L)] = rb1 + cx0
            idx_v[pl.ds(3 * B + off, L)] = rb1 + cx1
            w4 = pix * 4
            plsc.store_scatter(w_v, [w4], wy0 * wx0)
            plsc.store_scatter(w_v, [w4 + 1], wy0 * wx1)
            plsc.store_scatter(w_v, [w4 + 2], wy1 * wx0)
            plsc.store_scatter(w_v, [w4 + 3], wy1 * wx1)
        for seg in range(NSEG):
            pltpu.async_copy(tex_hbm.at[idx_v.at[pl.ds(seg * 128, 128)]],
                             rows_v.at[pl.ds(seg * 128, 128)], semg)

    def drain_gathers(idx_v, rows_v, semg):
        for seg in range(NSEG):
            pltpu.make_async_copy(
                tex_hbm.at[idx_v.at[pl.ds(seg * 128, 128)]],
                rows_v.at[pl.ds(seg * 128, 128)], semg).wait()

    def combine_out(i, w_v, rows_v):
        r = wid * ROWS_PER_W + i
        l512 = lanes * B

        @plsc.parallel_loop(0, B // 4, 1, unroll=2)
        def comb_body(p):
            wv = w_v[pl.ds(p * L, L)]
            for uu in range(4):
                q = p * 4 + uu
                v00 = rows_v[0 * B + q, :]
                v01 = rows_v[1 * B + q, :]
                v10 = rows_v[2 * B + q, :]
                v11 = rows_v[3 * B + q, :]
                acc = ((v00 * wv[uu * 4 + 0] + v01 * wv[uu * 4 + 1])
                       + (v10 * wv[uu * 4 + 2] + v11 * wv[uu * 4 + 3]))
                plsc.store_scatter(acc_v, [l512 + q], acc)

        ocopies = []
        for c in range(C):
            orow = r + (r // H) * (C - 1) * H + c * H
            ocopies.append(pltpu.async_copy(
                acc_v.at[pl.ds(c * B, B)], out_hbm.at[orow], semo))
        for cp in ocopies:
            cp.wait()

    fire_chunk(0, w0_v, idx0_v, rows0_v, semg0)

    def body(ii, carry):
        c0 = 2 * ii
        c1 = c0 + 1
        fire_chunk(c1, w1_v, idx1_v, rows1_v, semg1)
        drain_gathers(idx0_v, rows0_v, semg0)
        combine_out(c0, w0_v, rows0_v)
        cn = jnp.minimum(c0 + 2, ROWS_PER_W - 1)
        fire_chunk(cn, w0_v, idx0_v, rows0_v, semg0)
        drain_gathers(idx1_v, rows1_v, semg1)
        combine_out(c1, w1_v, rows1_v)
        return carry

    lax.fori_loop(0, ROWS_PER_W // 2, body, 0)
    drain_gathers(idx0_v, rows0_v, semg0)


@jax.jit
def kernel(uv_map, texture0):
    uv2 = uv_map.reshape(ROWS, 2 * W)
    tex2 = texture0.reshape(TEX * TEX, C)
    mesh = plsc.VectorSubcoreMesh(core_axis_name="c", subcore_axis_name="s")
    f = functools.partial(
        pl.kernel,
        out_type=jax.ShapeDtypeStruct((N * C * H, W), jnp.float32),
        mesh=mesh,
        compiler_params=pltpu.CompilerParams(needs_layout_passes=False,
                                             use_tc_tiling_on_sc=False),
        scratch_types=[
            pltpu.VMEM((2 * B,), jnp.float32),
            pltpu.VMEM((4 * B,), jnp.float32),
            pltpu.VMEM((4 * B,), jnp.float32),
            pltpu.VMEM((4 * B,), jnp.int32),
            pltpu.VMEM((4 * B,), jnp.int32),
            pltpu.VMEM((4 * B, C), jnp.float32),
            pltpu.VMEM((4 * B, C), jnp.float32),
            pltpu.VMEM((C * B,), jnp.float32),
            pltpu.SemaphoreType.DMA,
            pltpu.SemaphoreType.DMA,
            pltpu.SemaphoreType.DMA,
        ],
    )(_tex_kernel)
    out2 = f(uv2, tex2)
    return out2.reshape(N, C, H, W)


# speedup vs baseline: 4.2446x; 1.2727x over previous
import functools

import jax
import jax.numpy as jnp
from jax import lax
from jax.experimental import pallas as pl
from jax.experimental.pallas import tpu as pltpu
from jax.experimental.pallas import tpu_sc as plsc

NC, NS, L = 2, 16, 16
NW = NC * NS
TEX = 1024
C = 16
N, H, W = 4, 512, 512
ROWS = N * H
ROWS_PER_W = ROWS // NW
B = W
NG = B // L
NSUB = B // 128
NSEG = 4 * NSUB
Y_PER = TEX // NW


def _floor_i32(x):
    t = x.astype(jnp.int32)
    tf = t.astype(jnp.float32)
    return t - jnp.where(x < tf, 1, 0).astype(jnp.int32)


def _relayout_kernel(texr_hbm, ttab_hbm, blk0, blk1, tbl0, tbl1,
                     semi0, semi1, semo):
    wid = lax.axis_index("s") * NC + lax.axis_index("c")
    lanes = lax.iota(jnp.int32, L)

    def fire_in(y, blk, semi):
        pltpu.async_copy(texr_hbm.at[pl.ds(y * C, C)], blk, semi)

    def drain_in(y, blk, semi):
        pltpu.make_async_copy(texr_hbm.at[pl.ds(y * C, C)], blk, semi).wait()

    def work(blk, tbl):
        rbase = (lanes // 8) * 8
        cbase = (lanes - (lanes // 8) * 8) * 128
        for xt in range(8):
            rvec = rbase + xt

            @plsc.parallel_loop(0, 128, 1, unroll=4)
            def tbody(x1):
                col = plsc.load_gather(blk, [rvec, cbase + x1])
                tbl[xt * 128 + x1, :] = col

    def fire_out(y, tbl):
        pltpu.async_copy(tbl, ttab_hbm.at[pl.ds(y * TEX, TEX)], semo)

    def drain_out(y, tbl):
        pltpu.make_async_copy(tbl, ttab_hbm.at[pl.ds(y * TEX, TEX)],
                              semo).wait()

    y_base = wid * Y_PER
    fire_in(y_base, blk0, semi0)

    def body(ii, carry):
        y0 = y_base + 2 * ii
        y1 = y0 + 1
        fire_in(y1, blk1, semi1)
        drain_in(y0, blk0, semi0)
        work(blk0, tbl0)
        fire_out(y0, tbl0)
        yn = jnp.minimum(y0 + 2, y_base + Y_PER - 1)
        fire_in(yn, blk0, semi0)
        drain_in(y1, blk1, semi1)
        work(blk1, tbl1)
        fire_out(y1, tbl1)
        drain_out(y0, tbl0)
        drain_out(y1, tbl1)
        return carry

    lax.fori_loop(0, Y_PER // 2, body, 0)
    drain_in(y_base + Y_PER - 1, blk0, semi0)


def _tex_kernel(uv_hbm, tex_hbm, out_hbm, uv_v, w0_v, w1_v, idx0_v, idx1_v,
                rows0_v, rows1_v, acc_v, semg0, semg1, semo):
    wid = lax.axis_index("s") * NC + lax.axis_index("c")
    lanes = lax.iota(jnp.int32, L)

    def fire_chunk(i, w_v, idx_v, rows_v, semg):
        r = wid * ROWS_PER_W + i
        pltpu.sync_copy(uv_hbm.at[r], uv_v)

        @plsc.parallel_loop(0, NG, 1, unroll=2)
        def idx_body(g):
            pix = g * L + lanes
            j = g // 8
            boff = j * 256 + (g - j * 8) * L
            u = uv_v[pl.ds(boff, L)]
            v = uv_v[pl.ds(boff + 128, L)]
            gx = u * 2.0 - 1.0
            gy = -(v * 2.0 - 1.0)
            fx = ((gx + 1.0) * float(TEX) - 1.0) * 0.5
            fy = ((gy + 1.0) * float(TEX) - 1.0) * 0.5
            ix0 = _floor_i32(fx)
            iy0 = _floor_i32(fy)
            wx1 = fx - ix0.astype(jnp.float32)
            wy1 = fy - iy0.astype(jnp.float32)
            wx0 = 1.0 - wx1
            wy0 = 1.0 - wy1
            ix1 = ix0 + 1
            iy1 = iy0 + 1
            m = jnp.where(u == 0.0, 0.0, 1.0)
            wx0 = wx0 * m * jnp.where((ix0 >= 0) & (ix0 <= TEX - 1), 1.0, 0.0)
            wx1 = wx1 * m * jnp.where((ix1 >= 0) & (ix1 <= TEX - 1), 1.0, 0.0)
            wy0 = wy0 * jnp.where((iy0 >= 0) & (iy0 <= TEX - 1), 1.0, 0.0)
            wy1 = wy1 * jnp.where((iy1 >= 0) & (iy1 <= TEX - 1), 1.0, 0.0)
            cx0 = jnp.clip(ix0, 0, TEX - 1)
            cx1 = jnp.clip(ix1, 0, TEX - 1)
            rb0 = jnp.clip(iy0, 0, TEX - 1) * TEX
            rb1 = jnp.clip(iy1, 0, TEX - 1) * TEX
            off = g * L
            idx_v[pl.ds(0 * B + off, L)] = rb0 + cx0
            idx_v[pl.ds(1 * B + off, L)] = rb0 + cx1
            idx_v[pl.ds(2 * B + off, L)] = rb1 + cx0
            idx_v[pl.ds(3 * B + off, L)] = rb1 + cx1
            w4 = pix * 4
            plsc.store_scatter(w_v, [w4], wy0 * wx0)
            plsc.store_scatter(w_v, [w4 + 1], wy0 * wx1)
            plsc.store_scatter(w_v, [w4 + 2], wy1 * wx0)
            plsc.store_scatter(w_v, [w4 + 3], wy1 * wx1)

        for seg in range(NSEG):
            pltpu.async_copy(tex_hbm.at[idx_v.at[pl.ds(seg * 128, 128)]],
                             rows_v.at[pl.ds(seg * 128, 128)], semg)

    def drain_gathers(idx_v, rows_v, semg):
        for seg in range(NSEG):
            pltpu.make_async_copy(
                tex_hbm.at[idx_v.at[pl.ds(seg * 128, 128)]],
                rows_v.at[pl.ds(seg * 128, 128)], semg).wait()

    def combine_out(i, w_v, rows_v):
        r = wid * ROWS_PER_W + i
        l512 = lanes * B

        @plsc.parallel_loop(0, B // 4, 1, unroll=2)
        def comb_body(p):
            wv = w_v[pl.ds(p * L, L)]
            for uu in range(4):
                q = p * 4 + uu
                v00 = rows_v[0 * B + q, :]
                v01 = rows_v[1 * B + q, :]
                v10 = rows_v[2 * B + q, :]
                v11 = rows_v[3 * B + q, :]
                acc = ((v00 * wv[uu * 4 + 0] + v01 * wv[uu * 4 + 1])
                       + (v10 * wv[uu * 4 + 2] + v11 * wv[uu * 4 + 3]))
                plsc.store_scatter(acc_v, [l512 + q], acc)

        ocopies = []
        for c in range(C):
            orow = r + (r // H) * (C - 1) * H + c * H
            ocopies.append(pltpu.async_copy(
                acc_v.at[pl.ds(c * B, B)], out_hbm.at[orow], semo))
        for cp in ocopies:
            cp.wait()

    fire_chunk(0, w0_v, idx0_v, rows0_v, semg0)

    def body(ii, carry):
        c0 = 2 * ii
        c1 = c0 + 1
        fire_chunk(c1, w1_v, idx1_v, rows1_v, semg1)
        drain_gathers(idx0_v, rows0_v, semg0)
        combine_out(c0, w0_v, rows0_v)
        cn = jnp.minimum(c0 + 2, ROWS_PER_W - 1)
        fire_chunk(cn, w0_v, idx0_v, rows0_v, semg0)
        drain_gathers(idx1_v, rows1_v, semg1)
        combine_out(c1, w1_v, rows1_v)
        return carry

    lax.fori_loop(0, ROWS_PER_W // 2, body, 0)
    drain_gathers(idx0_v, rows0_v, semg0)


@jax.jit
def kernel(uv_map, texture0):
    uv2 = (uv_map.reshape(N, H, 4, 128, 2).transpose(0, 1, 2, 4, 3)
           .reshape(ROWS, 2 * W))
    texr = (texture0.reshape(TEX, 8, 128, 2, 8).transpose(0, 3, 1, 4, 2)
            .reshape(TEX * C, TEX))
    mesh = plsc.VectorSubcoreMesh(core_axis_name="c", subcore_axis_name="s")
    cp = pltpu.CompilerParams(needs_layout_passes=False,
                              use_tc_tiling_on_sc=False)

    relayout = functools.partial(
        pl.kernel,
        out_type=jax.ShapeDtypeStruct((TEX * TEX, C), jnp.float32),
        mesh=mesh,
        compiler_params=cp,
        scratch_types=[
            pltpu.VMEM((C, TEX), jnp.float32),
            pltpu.VMEM((C, TEX), jnp.float32),
            pltpu.VMEM((TEX, C), jnp.float32),
            pltpu.VMEM((TEX, C), jnp.float32),
            pltpu.SemaphoreType.DMA,
            pltpu.SemaphoreType.DMA,
            pltpu.SemaphoreType.DMA,
        ],
    )(_relayout_kernel)
    ttab = relayout(texr)

    sample = functools.partial(
        pl.kernel,
        out_type=jax.ShapeDtypeStruct((N * C * H, W), jnp.float32),
        mesh=mesh,
        compiler_params=cp,
        scratch_types=[
            pltpu.VMEM((2 * B,), jnp.float32),
            pltpu.VMEM((4 * B,), jnp.float32),
            pltpu.VMEM((4 * B,), jnp.float32),
            pltpu.VMEM((4 * B,), jnp.int32),
            pltpu.VMEM((4 * B,), jnp.int32),
            pltpu.VMEM((4 * B, C), jnp.float32),
            pltpu.VMEM((4 * B, C), jnp.float32),
            pltpu.VMEM((C * B,), jnp.float32),
            pltpu.SemaphoreType.DMA,
            pltpu.SemaphoreType.DMA,
            pltpu.SemaphoreType.DMA,
        ],
    )(_tex_kernel)
    out2 = sample(uv2, ttab)
    return out2.reshape(N, C, H, W)


# speedup vs baseline: 5.8402x; 1.3759x over previous
import functools

import jax
import jax.numpy as jnp
from jax import lax
from jax.experimental import pallas as pl
from jax.experimental.pallas import tpu as pltpu
from jax.experimental.pallas import tpu_sc as plsc

NC, NS, L = 2, 16, 16
NW = NC * NS
TEX = 1024
C = 16
N, H, W = 4, 512, 512
ROWS = N * H
ROWS_PER_W = ROWS // NW
B = W
NG = B // L
NSUB = B // 128
NSEG = 4 * NSUB
Y_PER = TEX // NW


def _floor_i32(x):
    t = x.astype(jnp.int32)
    tf = t.astype(jnp.float32)
    return t - jnp.where(x < tf, 1, 0).astype(jnp.int32)


def _relayout_kernel(texr_hbm, ttab_hbm, blk0, blk1, tbl0, tbl1,
                     semi0, semi1, semo):
    wid = lax.axis_index("s") * NC + lax.axis_index("c")
    lanes = lax.iota(jnp.int32, L)

    def fire_in(y, blk, semi):
        pltpu.async_copy(texr_hbm.at[pl.ds(y * C, C)], blk, semi)

    def drain_in(y, blk, semi):
        pltpu.make_async_copy(texr_hbm.at[pl.ds(y * C, C)], blk, semi).wait()

    def work(blk, tbl):
        rbase = (lanes // 8) * 8
        cbase = (lanes - (lanes // 8) * 8) * 128
        for xt in range(8):
            rvec = rbase + xt

            @plsc.parallel_loop(0, 128, 1, unroll=4)
            def tbody(x1):
                xv = (lanes + x1) & 127
                col = plsc.load_gather(blk, [rvec, cbase + xv])
                plsc.store_scatter(tbl, [xt * 128 + xv, lanes], col)

    def fire_out(y, tbl):
        pltpu.async_copy(tbl, ttab_hbm.at[pl.ds(y * TEX, TEX)], semo)

    def drain_out(y, tbl):
        pltpu.make_async_copy(tbl, ttab_hbm.at[pl.ds(y * TEX, TEX)],
                              semo).wait()

    y_base = wid * Y_PER
    fire_in(y_base, blk0, semi0)

    def body(ii, carry):
        y0 = y_base + 2 * ii
        y1 = y0 + 1
        fire_in(y1, blk1, semi1)
        drain_in(y0, blk0, semi0)
        work(blk0, tbl0)
        fire_out(y0, tbl0)
        yn = jnp.minimum(y0 + 2, y_base + Y_PER - 1)
        fire_in(yn, blk0, semi0)
        drain_in(y1, blk1, semi1)
        work(blk1, tbl1)
        fire_out(y1, tbl1)
        drain_out(y0, tbl0)
        drain_out(y1, tbl1)
        return carry

    lax.fori_loop(0, Y_PER // 2, body, 0)
    drain_in(y_base + Y_PER - 1, blk0, semi0)


def _tex_kernel(uv_hbm, tex_hbm, out_hbm, uv_v, w0_v, w1_v, idx0_v, idx1_v,
                rows0_v, rows1_v, acc_v, semg0, semg1, semo):
    wid = lax.axis_index("s") * NC + lax.axis_index("c")
    lanes = lax.iota(jnp.int32, L)

    def fire_chunk(i, w_v, idx_v, rows_v, semg):
        r = wid * ROWS_PER_W + i
        pltpu.sync_copy(uv_hbm.at[r], uv_v)

        @plsc.parallel_loop(0, NG, 1, unroll=2)
        def idx_body(g):
            pix = g * L + lanes
            j = g // 8
            boff = j * 256 + (g - j * 8) * L
            u = uv_v[pl.ds(boff, L)]
            v = uv_v[pl.ds(boff + 128, L)]
            gx = u * 2.0 - 1.0
            gy = -(v * 2.0 - 1.0)
            fx = ((gx + 1.0) * float(TEX) - 1.0) * 0.5
            fy = ((gy + 1.0) * float(TEX) - 1.0) * 0.5
            ix0 = _floor_i32(fx)
            iy0 = _floor_i32(fy)
            wx1 = fx - ix0.astype(jnp.float32)
            wy1 = fy - iy0.astype(jnp.float32)
            wx0 = 1.0 - wx1
            wy0 = 1.0 - wy1
            ix1 = ix0 + 1
            iy1 = iy0 + 1
            m = jnp.where(u == 0.0, 0.0, 1.0)
            wx0 = wx0 * m * jnp.where((ix0 >= 0) & (ix0 <= TEX - 1), 1.0, 0.0)
            wx1 = wx1 * m * jnp.where((ix1 >= 0) & (ix1 <= TEX - 1), 1.0, 0.0)
            wy0 = wy0 * jnp.where((iy0 >= 0) & (iy0 <= TEX - 1), 1.0, 0.0)
            wy1 = wy1 * jnp.where((iy1 >= 0) & (iy1 <= TEX - 1), 1.0, 0.0)
            cx0 = jnp.clip(ix0, 0, TEX - 1)
            cx1 = jnp.clip(ix1, 0, TEX - 1)
            rb0 = jnp.clip(iy0, 0, TEX - 1) * TEX
            rb1 = jnp.clip(iy1, 0, TEX - 1) * TEX
            off = g * L
            idx_v[pl.ds(0 * B + off, L)] = rb0 + cx0
            idx_v[pl.ds(1 * B + off, L)] = rb0 + cx1
            idx_v[pl.ds(2 * B + off, L)] = rb1 + cx0
            idx_v[pl.ds(3 * B + off, L)] = rb1 + cx1
            w4 = pix * 4
            plsc.store_scatter(w_v, [w4], wy0 * wx0)
            plsc.store_scatter(w_v, [w4 + 1], wy0 * wx1)
            plsc.store_scatter(w_v, [w4 + 2], wy1 * wx0)
            plsc.store_scatter(w_v, [w4 + 3], wy1 * wx1)

        for seg in range(NSEG):
            pltpu.async_copy(tex_hbm.at[idx_v.at[pl.ds(seg * 128, 128)]],
                             rows_v.at[pl.ds(seg * 128, 128)], semg)

    def drain_gathers(idx_v, rows_v, semg):
        for seg in range(NSEG):
            pltpu.make_async_copy(
                tex_hbm.at[idx_v.at[pl.ds(seg * 128, 128)]],
                rows_v.at[pl.ds(seg * 128, 128)], semg).wait()

    def combine_out(i, w_v, rows_v):
        r = wid * ROWS_PER_W + i
        l4 = lanes * 4

        @plsc.parallel_loop(0, B // 4, 1, unroll=2)
        def comb_body(p):
            wv = w_v[pl.ds(p * L, L)]
            for uu in range(4):
                q = p * 4 + uu
                v00 = rows_v[0 * B + q, :]
                v01 = rows_v[1 * B + q, :]
                v10 = rows_v[2 * B + q, :]
                v11 = rows_v[3 * B + q, :]
                acc = ((v00 * wv[uu * 4 + 0] + v01 * wv[uu * 4 + 1])
                       + (v10 * wv[uu * 4 + 2] + v11 * wv[uu * 4 + 3]))
                qd = q // 128
                plsc.store_scatter(
                    acc_v, [l4 + qd, jnp.zeros((L,), jnp.int32) + (q - qd * 128)],
                    acc)

        n = r // H
        h = r - n * H
        ht = h // 8
        h8 = h - ht * 8
        ocopies = []
        for c in range(C):
            orow = n * 4096 + c * 256 + ht * 4
            ocopies.append(pltpu.async_copy(
                acc_v.at[pl.ds(c * 4, 4)],
                out_hbm.at[pl.ds(orow, 4), pl.ds(h8 * 128, 128)], semo))
        for cp in ocopies:
            cp.wait()

    fire_chunk(0, w0_v, idx0_v, rows0_v, semg0)

    def body(ii, carry):
        c0 = 2 * ii
        c1 = c0 + 1
        fire_chunk(c1, w1_v, idx1_v, rows1_v, semg1)
        drain_gathers(idx0_v, rows0_v, semg0)
        combine_out(c0, w0_v, rows0_v)
        cn = jnp.minimum(c0 + 2, ROWS_PER_W - 1)
        fire_chunk(cn, w0_v, idx0_v, rows0_v, semg0)
        drain_gathers(idx1_v, rows1_v, semg1)
        combine_out(c1, w1_v, rows1_v)
        return carry

    lax.fori_loop(0, ROWS_PER_W // 2, body, 0)
    drain_gathers(idx0_v, rows0_v, semg0)


@jax.jit
def kernel(uv_map, texture0):
    uv2 = (uv_map.reshape(N, H, 4, 128, 2).transpose(0, 1, 2, 4, 3)
           .reshape(ROWS, 2 * W))
    texr = (texture0.reshape(TEX, 8, 128, 2, 8).transpose(0, 3, 1, 4, 2)
            .reshape(TEX * C, TEX))
    mesh = plsc.VectorSubcoreMesh(core_axis_name="c", subcore_axis_name="s")
    cp = pltpu.CompilerParams(needs_layout_passes=False,
                              use_tc_tiling_on_sc=False)

    relayout = functools.partial(
        pl.kernel,
        out_type=jax.ShapeDtypeStruct((TEX * TEX, C), jnp.float32),
        mesh=mesh,
        compiler_params=cp,
        scratch_types=[
            pltpu.VMEM((C, TEX), jnp.float32),
            pltpu.VMEM((C, TEX), jnp.float32),
            pltpu.VMEM((TEX, C), jnp.float32),
            pltpu.VMEM((TEX, C), jnp.float32),
            pltpu.SemaphoreType.DMA,
            pltpu.SemaphoreType.DMA,
            pltpu.SemaphoreType.DMA,
        ],
    )(_relayout_kernel)
    ttab = relayout(texr)

    sample = functools.partial(
        pl.kernel,
        out_type=jax.ShapeDtypeStruct((N * C * H * W // 1024, 1024),
                                      jnp.float32),
        mesh=mesh,
        compiler_params=cp,
        scratch_types=[
            pltpu.VMEM((2 * B,), jnp.float32),
            pltpu.VMEM((4 * B,), jnp.float32),
            pltpu.VMEM((4 * B,), jnp.float32),
            pltpu.VMEM((4 * B,), jnp.int32),
            pltpu.VMEM((4 * B,), jnp.int32),
            pltpu.VMEM((4 * B, C), jnp.float32),
            pltpu.VMEM((4 * B, C), jnp.float32),
            pltpu.VMEM((C * 4, 128), jnp.float32),
            pltpu.SemaphoreType.DMA,
            pltpu.SemaphoreType.DMA,
            pltpu.SemaphoreType.DMA,
        ],
    )(_tex_kernel)
    out2 = sample(uv2, ttab)
    return (out2.reshape(N, C, H // 8, 4, 8, 128).transpose(0, 1, 2, 4, 3, 5)
            .reshape(N, C, H, W))
